# baseline (speedup 1.0000x reference)
import functools

import numpy as np
import jax
import jax.numpy as jnp
from jax import lax
from jax.experimental import pallas as pl
from jax.experimental.pallas import tpu as pltpu
from jax.experimental.pallas import tpu_sc as plsc

_N, _M, _SD, _CD, _OD = 64, 32768, 16, 8, 8
_LOGM = np.float32(np.log(_M))
_TINY = np.float32(np.finfo(np.float32).tiny)

_ROTS = ((13, 15, 26, 6), (17, 29, 16, 24), (13, 15, 26, 6),
         (17, 29, 16, 24), (13, 15, 26, 6))


def _threefry_consts(seed):
  ks0 = np.uint32(0)
  ks1 = np.uint32(seed)
  ks2 = np.uint32(np.uint32(0) ^ np.uint32(seed) ^ np.uint32(0x1BD11BDA))
  inj = ((ks1, np.uint32(ks2 + np.uint32(1))),
         (ks2, np.uint32(ks0 + np.uint32(2))),
         (ks0, np.uint32(ks1 + np.uint32(3))),
         (ks1, np.uint32(ks2 + np.uint32(4))),
         (ks2, np.uint32(ks0 + np.uint32(5))))
  return ks0, ks1, inj


def _rotl(x, r):
  return (x << np.uint32(r)) | (x >> np.uint32(32 - r))


def _threefry_bits(x0, x1, inj):
  for g in range(5):
    for r in _ROTS[g]:
      x0 = x0 + x1
      x1 = _rotl(x1, r) ^ x0
    a, b = inj[g]
    if a:
      x0 = x0 + a
    if b:
      x1 = x1 + b
  return x0 ^ x1


def _u01(bits):
  fb = (bits >> np.uint32(9)) | np.uint32(0x3F800000)
  return lax.bitcast_convert_type(fb, jnp.float32) - np.float32(1.0)


def _bf_rtne(x):
  b = lax.bitcast_convert_type(x, jnp.uint32)
  r = (b + np.uint32(0x7FFF) + ((b >> np.uint32(16)) & np.uint32(1))) \
      & np.uint32(0xFFFF0000)
  return lax.bitcast_convert_type(r, jnp.float32)


def _dot_bf16_tree(x, w_ref, k):
  xb = _bf_rtne(x)
  terms = [xb[:, s:s + 1] * _bf_rtne(w_ref[s:s + 1, :]) for s in range(k)]
  while len(terms) > 1:
    terms = [terms[i] + terms[i + 1] for i in range(0, len(terms), 2)]
  return terms[0]


_BMA = 2048


def _stage1_body(sp_ref, lw_ref, obs_ref, ctl_ref, at_ref, bt_ref, wt_ref,
                 bm_ref, spred_ref, lwun_ref, est_ref, lwc_ref,
                 mx_s, z_s, ws_s):
  n = pl.program_id(0)
  mb = pl.program_id(1)
  nmb = pl.num_programs(1)

  sp = sp_ref[0]

  acc = _dot_bf16_tree(sp, at_ref, _SD)

  cacc = _dot_bf16_tree(ctl_ref[0], bt_ref, _CD)

  ks0, ks1, inj = _threefry_consts(1234)
  base = (jnp.uint32(n) << np.uint32(19)) + \
      (jnp.uint32(mb) * np.uint32(_BMA * _SD))
  row = lax.broadcasted_iota(jnp.uint32, (_BMA, _SD), 0)
  col = lax.broadcasted_iota(jnp.uint32, (_BMA, _SD), 1)
  lo = (row << np.uint32(4)) + col + (base + ks1)
  bits = _threefry_bits(jnp.full((_BMA, _SD), ks0, jnp.uint32), lo, inj)
  flo = _u01(bits)
  ulo = np.float32(np.nextafter(np.float32(-1.0), np.float32(0.0)))
  udiff = np.float32(np.float32(1.0) - ulo)
  u = jnp.maximum(ulo, flo * udiff + ulo)
  z = np.float32(np.sqrt(2.0)) * lax.erf_inv(u)
  noise = np.float32(0.05) * z

  spred = (acc + cacc) + noise
  spred_ref[0] = spred

  pred_obs = _dot_bf16_tree(spred, wt_ref, _SD) + bm_ref[0:1, :]
  d = pred_obs - obs_ref[0]
  oll = np.float32(-0.5) * jnp.sum(d * d, axis=1)

  lw_un = lw_ref[0, 0] + oll
  lwun_ref[0, 0] = lw_un
  lwc_ref[0, 0] = jnp.full((_BMA,), -_LOGM, jnp.float32)

  @pl.when(mb == 0)
  def _init():
    mx_s[0] = np.float32(-np.inf)
    z_s[0] = np.float32(0.0)
    ws_s[0, :] = jnp.zeros((_SD,), jnp.float32)

  bmax = jnp.max(lw_un)
  old_mx = mx_s[0]
  new_mx = jnp.maximum(old_mx, bmax)
  scale = jnp.exp(old_mx - new_mx)
  e = jnp.exp(lw_un - new_mx)
  z_s[0] = z_s[0] * scale + jnp.sum(e)
  ws_s[0, :] = ws_s[0, :] * scale + jnp.sum(e[:, None] * spred, axis=0)
  mx_s[0] = new_mx

  @pl.when(mb == nmb - 1)
  def _fin():
    est_ref[0, 0, :] = ws_s[0, :] / z_s[0]


def _stage1(states_prev, lw_prev, obs, controls, a_t, b_t, w_t, b_meas):
  grid = (_N, _M // _BMA)
  return pl.pallas_call(
      _stage1_body,
      grid=grid,
      in_specs=[
          pl.BlockSpec((1, _BMA, _SD), lambda n, mb: (n, mb, 0)),
          pl.BlockSpec((1, 1, _BMA), lambda n, mb: (n, 0, mb)),
          pl.BlockSpec((1, 1, _OD), lambda n, mb: (n, 0, 0)),
          pl.BlockSpec((1, 1, _CD), lambda n, mb: (n, 0, 0)),
          pl.BlockSpec((_SD, _SD), lambda n, mb: (0, 0)),
          pl.BlockSpec((_CD, _SD), lambda n, mb: (0, 0)),
          pl.BlockSpec((_SD, _OD), lambda n, mb: (0, 0)),
          pl.BlockSpec((1, _OD), lambda n, mb: (0, 0)),
      ],
      out_specs=[
          pl.BlockSpec((1, _BMA, _SD), lambda n, mb: (n, mb, 0)),
          pl.BlockSpec((1, 1, _BMA), lambda n, mb: (n, 0, mb)),
          pl.BlockSpec((1, 1, _SD), lambda n, mb: (n, 0, 0)),
          pl.BlockSpec((1, 1, _BMA), lambda n, mb: (n, 0, mb)),
      ],
      out_shape=[
          jax.ShapeDtypeStruct((_N, _M, _SD), jnp.float32),
          jax.ShapeDtypeStruct((_N, 1, _M), jnp.float32),
          jax.ShapeDtypeStruct((_N, 1, _SD), jnp.float32),
          jax.ShapeDtypeStruct((_N, 1, _M), jnp.float32),
      ],
      scratch_shapes=[
          pltpu.SMEM((1,), jnp.float32),
          pltpu.SMEM((1,), jnp.float32),
          pltpu.VMEM((1, _SD), jnp.float32),
      ],
      compiler_params=pltpu.CompilerParams(
          dimension_semantics=("parallel", "arbitrary")),
  )(states_prev, lw_prev, obs, controls, a_t, b_t, w_t, b_meas)


_TS, _TL = 8, 1024
_NT = _M // (_TS * _TL)


def _stage2_body(lw_smem, out_ref):
  n = pl.program_id(0)
  _, ks1, inj = _threefry_consts(5678)
  base_n = (n << 15).astype(jnp.int32)

  for tile in range(_NT):
    tbase = np.uint32(tile * _TS * _TL)
    s_io = lax.broadcasted_iota(jnp.uint32, (_TS, _TL), 0)
    l_io = lax.broadcasted_iota(jnp.uint32, (_TS, _TL), 1)
    m_vec = tbase + s_io * np.uint32(_TL) + l_io
    hi_vec = m_vec >> np.uint32(11)
    lo_base = ((m_vec & np.uint32(2047)) << np.uint32(21)) | \
        (jnp.uint32(n) << np.uint32(15))
    lo_base = lo_base + ks1

    neg_inf = jnp.full((_TS, _TL), -np.inf, jnp.float32)
    zero_j = jnp.zeros((_TS, _TL), jnp.int32)

    def jbody(j, carry, hi_vec=hi_vec, lo_base=lo_base):
      best, bj = carry
      bits = _threefry_bits(hi_vec, lo_base + jnp.uint32(j), inj)
      flo = _u01(bits)
      u = jnp.maximum(_TINY, flo + _TINY)
      g = -jnp.log(-jnp.log(u))
      t = g + lw_smem[0, 0, j]
      pred = t > best
      best = jnp.where(pred, t, best)
      bj = jnp.where(pred, jnp.full((_TS, _TL), 1, jnp.int32) * j + base_n, bj)
      return best, bj

    _, bj = lax.fori_loop(0, _M, jbody, (neg_inf, zero_j))
    out_ref[0, tile] = bj


def _stage2(lw_un):
  return pl.pallas_call(
      _stage2_body,
      grid=(_N,),
      in_specs=[
          pl.BlockSpec((1, 1, _M), lambda n: (n, 0, 0),
                       memory_space=pltpu.SMEM),
      ],
      out_specs=[
          pl.BlockSpec((1, _NT, _TS, _TL), lambda n: (n, 0, 0, 0)),
      ],
      out_shape=[
          jax.ShapeDtypeStruct((_N, _NT, _TS, _TL), jnp.int32),
      ],
      compiler_params=pltpu.CompilerParams(
          dimension_semantics=("parallel",)),
  )(lw_un)[0]


_GCH = 2048


def _stage3(sp_flat, idx_flat):
  info = plsc.get_sparse_core_info()
  nw = info.num_cores * info.num_subcores
  b = _N * _M
  b_per_w = b // nw
  mesh = plsc.VectorSubcoreMesh(core_axis_name="c", subcore_axis_name="s")

  @functools.partial(
      pl.kernel, mesh=mesh,
      out_type=jax.ShapeDtypeStruct((b, _SD), jnp.float32),
      compiler_params=pltpu.CompilerParams(use_tc_tiling_on_sc=False),
      scratch_types=[
          pltpu.VMEM((_GCH,), jnp.int32),
          pltpu.VMEM((_GCH, _SD), jnp.float32),
          pltpu.SemaphoreType.DMA,
      ],
  )
  def k(sp_hbm, idx_hbm, out_hbm, idx_v, rows_v, sem):
    wid = lax.axis_index("s") * info.num_cores + lax.axis_index("c")
    base = wid * b_per_w

    def body(i, _):
      off = base + i * _GCH
      pltpu.sync_copy(idx_hbm.at[pl.ds(off, _GCH)], idx_v)
      pltpu.async_copy(sp_hbm.at[idx_v], rows_v, sem).wait()
      pltpu.sync_copy(rows_v, out_hbm.at[pl.ds(off, _GCH)])
      return 0

    lax.fori_loop(0, b_per_w // _GCH, body, 0)

  return k(sp_flat, idx_flat)


@jax.jit
def kernel(states_prev, log_weights_prev, observations, controls,
           A_dyn, B_dyn, W_meas, b_meas):
  a_t = A_dyn.T
  b_t = B_dyn.T
  w_t = W_meas.T
  bm = b_meas.reshape(1, _OD)

  states_pred, lw_un, est, log_weights = _stage1(
      states_prev, log_weights_prev.reshape(_N, 1, _M),
      observations.reshape(_N, 1, _OD), controls.reshape(_N, 1, _CD),
      a_t, b_t, w_t, bm)

  idx = _stage2(lw_un)

  sp_flat = states_pred.reshape(_N * _M, _SD)
  idx_flat = idx.reshape(_N * _M)
  states = _stage3(sp_flat, idx_flat).reshape(_N, _M, _SD)

  return est.reshape(_N, _SD), states, log_weights.reshape(_N, _M)


# speedup vs baseline: 1.1564x; 1.1564x over previous
import functools

import numpy as np
import jax
import jax.numpy as jnp
from jax import lax
from jax.experimental import pallas as pl
from jax.experimental.pallas import tpu as pltpu
from jax.experimental.pallas import tpu_sc as plsc

_N, _M, _SD, _CD, _OD = 64, 32768, 16, 8, 8
_LOGM = np.float32(np.log(_M))
_TINY = np.float32(np.finfo(np.float32).tiny)

_ROTS = ((13, 15, 26, 6), (17, 29, 16, 24), (13, 15, 26, 6),
         (17, 29, 16, 24), (13, 15, 26, 6))


def _threefry_consts(seed):
  ks0 = np.uint32(0)
  ks1 = np.uint32(seed)
  ks2 = np.uint32(np.uint32(0) ^ np.uint32(seed) ^ np.uint32(0x1BD11BDA))
  inj = ((ks1, np.uint32(ks2 + np.uint32(1))),
         (ks2, np.uint32(ks0 + np.uint32(2))),
         (ks0, np.uint32(ks1 + np.uint32(3))),
         (ks1, np.uint32(ks2 + np.uint32(4))),
         (ks2, np.uint32(ks0 + np.uint32(5))))
  return ks0, ks1, inj


def _rotl(x, r):
  return (x << np.uint32(r)) | (x >> np.uint32(32 - r))


def _threefry_bits(x0, x1, inj):
  for g in range(5):
    for r in _ROTS[g]:
      x0 = x0 + x1
      x1 = _rotl(x1, r) ^ x0
    a, b = inj[g]
    if a:
      x0 = x0 + a
    if b:
      x1 = x1 + b
  return x0 ^ x1


def _u01(bits):
  fb = (bits >> np.uint32(9)) | np.uint32(0x3F800000)
  return lax.bitcast_convert_type(fb, jnp.float32) - np.float32(1.0)


def _bf_rtne(x):
  b = lax.bitcast_convert_type(x, jnp.uint32)
  r = (b + np.uint32(0x7FFF) + ((b >> np.uint32(16)) & np.uint32(1))) \
      & np.uint32(0xFFFF0000)
  return lax.bitcast_convert_type(r, jnp.float32)


def _dot_bf16_tree(x, w_ref, k):
  xb = _bf_rtne(x)
  terms = [xb[:, s:s + 1] * _bf_rtne(w_ref[s:s + 1, :]) for s in range(k)]
  while len(terms) > 1:
    terms = [terms[i] + terms[i + 1] for i in range(0, len(terms), 2)]
  return terms[0]


_BMA = 2048


def _stage1_body(sp_ref, lw_ref, obs_ref, ctl_ref, at_ref, bt_ref, wt_ref,
                 bm_ref, spred_ref, lwun_ref, est_ref, lwc_ref, mxo_ref,
                 mx_s, z_s, ws_s):
  n = pl.program_id(0)
  mb = pl.program_id(1)
  nmb = pl.num_programs(1)

  sp = sp_ref[0]

  acc = _dot_bf16_tree(sp, at_ref, _SD)

  cacc = _dot_bf16_tree(ctl_ref[0], bt_ref, _CD)

  ks0, ks1, inj = _threefry_consts(1234)
  base = (jnp.uint32(n) << np.uint32(19)) + \
      (jnp.uint32(mb) * np.uint32(_BMA * _SD))
  row = lax.broadcasted_iota(jnp.uint32, (_BMA, _SD), 0)
  col = lax.broadcasted_iota(jnp.uint32, (_BMA, _SD), 1)
  lo = (row << np.uint32(4)) + col + (base + ks1)
  bits = _threefry_bits(jnp.full((_BMA, _SD), ks0, jnp.uint32), lo, inj)
  flo = _u01(bits)
  ulo = np.float32(np.nextafter(np.float32(-1.0), np.float32(0.0)))
  udiff = np.float32(np.float32(1.0) - ulo)
  u = jnp.maximum(ulo, flo * udiff + ulo)
  z = np.float32(np.sqrt(2.0)) * lax.erf_inv(u)
  noise = np.float32(0.05) * z

  spred = (acc + cacc) + noise
  spred_ref[0] = spred

  pred_obs = _dot_bf16_tree(spred, wt_ref, _SD) + bm_ref[0:1, :]
  d = pred_obs - obs_ref[0]
  oll = np.float32(-0.5) * jnp.sum(d * d, axis=1)

  lw_un = lw_ref[0, 0] + oll
  lwun_ref[0, 0] = lw_un
  lwc_ref[0, 0] = jnp.full((_BMA,), -_LOGM, jnp.float32)

  @pl.when(mb == 0)
  def _init():
    mx_s[0] = np.float32(-np.inf)
    z_s[0] = np.float32(0.0)
    ws_s[0, :] = jnp.zeros((_SD,), jnp.float32)

  bmax = jnp.max(lw_un)
  old_mx = mx_s[0]
  new_mx = jnp.maximum(old_mx, bmax)
  scale = jnp.exp(old_mx - new_mx)
  e = jnp.exp(lw_un - new_mx)
  z_s[0] = z_s[0] * scale + jnp.sum(e)
  ws_s[0, :] = ws_s[0, :] * scale + jnp.sum(e[:, None] * spred, axis=0)
  mx_s[0] = new_mx

  @pl.when(mb == nmb - 1)
  def _fin():
    est_ref[0, 0, :] = ws_s[0, :] / z_s[0]
    mxo_ref[0, 0, 0] = mx_s[0]


def _stage1(states_prev, lw_prev, obs, controls, a_t, b_t, w_t, b_meas):
  grid = (_N, _M // _BMA)
  return pl.pallas_call(
      _stage1_body,
      grid=grid,
      in_specs=[
          pl.BlockSpec((1, _BMA, _SD), lambda n, mb: (n, mb, 0)),
          pl.BlockSpec((1, 1, _BMA), lambda n, mb: (n, 0, mb)),
          pl.BlockSpec((1, 1, _OD), lambda n, mb: (n, 0, 0)),
          pl.BlockSpec((1, 1, _CD), lambda n, mb: (n, 0, 0)),
          pl.BlockSpec((_SD, _SD), lambda n, mb: (0, 0)),
          pl.BlockSpec((_CD, _SD), lambda n, mb: (0, 0)),
          pl.BlockSpec((_SD, _OD), lambda n, mb: (0, 0)),
          pl.BlockSpec((1, _OD), lambda n, mb: (0, 0)),
      ],
      out_specs=[
          pl.BlockSpec((1, _BMA, _SD), lambda n, mb: (n, mb, 0)),
          pl.BlockSpec((1, 1, _BMA), lambda n, mb: (n, 0, mb)),
          pl.BlockSpec((1, 1, _SD), lambda n, mb: (n, 0, 0)),
          pl.BlockSpec((1, 1, _BMA), lambda n, mb: (n, 0, mb)),
          pl.BlockSpec((1, 1, 1), lambda n, mb: (n, 0, 0),
                       memory_space=pltpu.SMEM),
      ],
      out_shape=[
          jax.ShapeDtypeStruct((_N, _M, _SD), jnp.float32),
          jax.ShapeDtypeStruct((_N, 1, _M), jnp.float32),
          jax.ShapeDtypeStruct((_N, 1, _SD), jnp.float32),
          jax.ShapeDtypeStruct((_N, 1, _M), jnp.float32),
          jax.ShapeDtypeStruct((_N, 1, 1), jnp.float32),
      ],
      scratch_shapes=[
          pltpu.SMEM((1,), jnp.float32),
          pltpu.SMEM((1,), jnp.float32),
          pltpu.VMEM((1, _SD), jnp.float32),
      ],
      compiler_params=pltpu.CompilerParams(
          dimension_semantics=("parallel", "arbitrary")),
  )(states_prev, lw_prev, obs, controls, a_t, b_t, w_t, b_meas)


def _winv_body(lw_ref, mx_ref, winv_ref):
  winv_ref[0, 0] = jnp.exp(mx_ref[0, 0, 0] - lw_ref[0, 0])


def _winv(lw_un, rowmax):
  return pl.pallas_call(
      _winv_body,
      grid=(_N, _M // _BMA),
      in_specs=[
          pl.BlockSpec((1, 1, _BMA), lambda n, mb: (n, 0, mb)),
          pl.BlockSpec((1, 1, 1), lambda n, mb: (n, 0, 0),
                       memory_space=pltpu.SMEM),
      ],
      out_specs=[pl.BlockSpec((1, 1, _BMA), lambda n, mb: (n, 0, mb))],
      out_shape=[jax.ShapeDtypeStruct((_N, 1, _M), jnp.float32)],
      compiler_params=pltpu.CompilerParams(
          dimension_semantics=("parallel", "arbitrary")),
  )(lw_un, rowmax)[0]


_TS, _TL = 8, 1024
_NT = _M // (_TS * _TL)


def _stage2_body(lw_smem, out_ref):
  n = pl.program_id(0)
  _, ks1, inj = _threefry_consts(5678)
  base_n = (n << 15).astype(jnp.int32)

  for tile in range(_NT):
    tbase = np.uint32(tile * _TS * _TL)
    s_io = lax.broadcasted_iota(jnp.uint32, (_TS, _TL), 0)
    l_io = lax.broadcasted_iota(jnp.uint32, (_TS, _TL), 1)
    m_vec = tbase + s_io * np.uint32(_TL) + l_io
    hi_vec = m_vec >> np.uint32(11)
    lo_base = ((m_vec & np.uint32(2047)) << np.uint32(21)) | \
        (jnp.uint32(n) << np.uint32(15))
    lo_base = lo_base + ks1

    neg_inf = jnp.full((_TS, _TL), -np.inf, jnp.float32)
    zero_j = jnp.zeros((_TS, _TL), jnp.int32)

    def jbody(j, carry, hi_vec=hi_vec, lo_base=lo_base):
      best, bj = carry
      bits = _threefry_bits(hi_vec, lo_base + jnp.uint32(j), inj)
      flo = _u01(bits)
      u = flo + _TINY
      val = jnp.log(u) * lw_smem[0, 0, j]
      pred = val > best
      best = jnp.maximum(best, val)
      bj = jnp.where(pred, jnp.full((_TS, _TL), 1, jnp.int32) * j + base_n, bj)
      return best, bj

    _, bj = lax.fori_loop(0, _M, jbody, (neg_inf, zero_j), unroll=16)
    out_ref[0, tile] = bj


def _stage2(lw_un):
  return pl.pallas_call(
      _stage2_body,
      grid=(_N,),
      in_specs=[
          pl.BlockSpec((1, 1, _M), lambda n: (n, 0, 0),
                       memory_space=pltpu.SMEM),
      ],
      out_specs=[
          pl.BlockSpec((1, _NT, _TS, _TL), lambda n: (n, 0, 0, 0)),
      ],
      out_shape=[
          jax.ShapeDtypeStruct((_N, _NT, _TS, _TL), jnp.int32),
      ],
      compiler_params=pltpu.CompilerParams(
          dimension_semantics=("parallel",)),
  )(lw_un)[0]


_GCH = 2048


def _stage3(sp_flat, idx_flat):
  info = plsc.get_sparse_core_info()
  nw = info.num_cores * info.num_subcores
  b = _N * _M
  b_per_w = b // nw
  mesh = plsc.VectorSubcoreMesh(core_axis_name="c", subcore_axis_name="s")

  @functools.partial(
      pl.kernel, mesh=mesh,
      out_type=jax.ShapeDtypeStruct((b, _SD), jnp.float32),
      compiler_params=pltpu.CompilerParams(use_tc_tiling_on_sc=False),
      scratch_types=[
          pltpu.VMEM((_GCH,), jnp.int32),
          pltpu.VMEM((_GCH, _SD), jnp.float32),
          pltpu.SemaphoreType.DMA,
      ],
  )
  def k(sp_hbm, idx_hbm, out_hbm, idx_v, rows_v, sem):
    wid = lax.axis_index("s") * info.num_cores + lax.axis_index("c")
    base = wid * b_per_w

    def body(i, _):
      off = base + i * _GCH
      pltpu.sync_copy(idx_hbm.at[pl.ds(off, _GCH)], idx_v)
      pltpu.async_copy(sp_hbm.at[idx_v], rows_v, sem).wait()
      pltpu.sync_copy(rows_v, out_hbm.at[pl.ds(off, _GCH)])
      return 0

    lax.fori_loop(0, b_per_w // _GCH, body, 0)

  return k(sp_flat, idx_flat)


@jax.jit
def kernel(states_prev, log_weights_prev, observations, controls,
           A_dyn, B_dyn, W_meas, b_meas):
  a_t = A_dyn.T
  b_t = B_dyn.T
  w_t = W_meas.T
  bm = b_meas.reshape(1, _OD)

  states_pred, lw_un, est, log_weights, rowmax = _stage1(
      states_prev, log_weights_prev.reshape(_N, 1, _M),
      observations.reshape(_N, 1, _OD), controls.reshape(_N, 1, _CD),
      a_t, b_t, w_t, bm)

  winv = _winv(lw_un, rowmax)
  idx = _stage2(winv)

  sp_flat = states_pred.reshape(_N * _M, _SD)
  idx_flat = idx.reshape(_N * _M)
  states = _stage3(sp_flat, idx_flat).reshape(_N, _M, _SD)

  return est.reshape(_N, _SD), states, log_weights.reshape(_N, _M)
